# baseline (speedup 1.0000x reference)
import functools

import jax
import jax.numpy as jnp
from jax import lax
from jax.experimental import pallas as pl
from jax.experimental.pallas import tpu as pltpu
from jax.experimental.pallas import tpu_sc as plsc

B = 4096
L = 200
E = 4
NLANE = 16
COLS = tuple(NLANE * j for j in range(L // NLANE - 1)) + (L - NLANE,)

NC = 2
NS = 16
NW = NC * NS

ROWS_PER_WORKER = B // NW
ROWS_PER_CHUNK = 16
CHUNKS = ROWS_PER_WORKER // ROWS_PER_CHUNK
CHUNK_IDX = ROWS_PER_CHUNK * L


def _sc_embed(x_flat, luts):
    mesh = plsc.VectorSubcoreMesh(core_axis_name="c", subcore_axis_name="s")

    @functools.partial(
        pl.kernel,
        mesh=mesh,
        compiler_params=pltpu.CompilerParams(needs_layout_passes=False),
        out_type=jax.ShapeDtypeStruct((2 * B, E, L), jnp.float32),
        scratch_types=[
            pltpu.VMEM((CHUNK_IDX,), jnp.int32),
            pltpu.VMEM((ROWS_PER_CHUNK, E, L), jnp.float32),
            pltpu.VMEM((ROWS_PER_CHUNK, E, L), jnp.float32),
            [pltpu.VMEM((8 * NLANE,), jnp.float32) for _ in range(2 * E)],
        ],
    )
    def run(x_hbm, luts_hbm, out_hbm, xbuf, obf, obr, lutv):
        wid = lax.axis_index("s") * NC + lax.axis_index("c")
        for i in range(2 * E):
            pltpu.sync_copy(luts_hbm.at[i], lutv[i])
        lane = lax.iota(jnp.int32, NLANE)

        def chunk_body(c, carry):
            base_row = wid * ROWS_PER_WORKER + c * ROWS_PER_CHUNK
            pltpu.sync_copy(x_hbm.at[pl.ds(base_row * L, CHUNK_IDX)], xbuf)

            @plsc.parallel_loop(0, ROWS_PER_CHUNK, unroll=2)
            def row_body(r):
                rb_in = r * L
                for col in COLS:
                    xa = xbuf[pl.ds(rb_in + col, NLANE)]
                    xb = xbuf[pl.ds(rb_in + (L - NLANE) - col, NLANE)]
                    xa = (xa << 4) + lane
                    xr = (lax.rev(xb, (0,)) << 4) + lane
                    for e in range(E):
                        obf[r, e, pl.ds(col, NLANE)] = plsc.load_gather(
                            lutv[e], [xa]
                        )
                        obr[r, e, pl.ds(col, NLANE)] = plsc.load_gather(
                            lutv[E + e], [xr]
                        )

            pltpu.sync_copy(obf, out_hbm.at[pl.ds(base_row, ROWS_PER_CHUNK)])
            pltpu.sync_copy(
                obr, out_hbm.at[pl.ds(B + base_row, ROWS_PER_CHUNK)]
            )
            return carry

        lax.fori_loop(0, CHUNKS, chunk_body, 0)

    return run(x_flat, luts)


def kernel(x, weight, weight_rc):
    x_flat = x.reshape(-1).astype(jnp.int32)
    luts = jnp.zeros((2 * E, 8), jnp.float32)
    luts = luts.at[:E, : E + 1].set(weight.T)
    luts = luts.at[E:, : E + 1].set(weight_rc.T)
    luts = jnp.repeat(luts, NLANE, axis=1)
    return _sc_embed(x_flat, luts)
